# Initial kernel scaffold; baseline (speedup 1.0000x reference)
import jax
import jax.numpy as jnp
from jax.experimental import pallas as pl


def kernel(x, table, W, b):
    raise NotImplementedError("write your pallas kernel here")


# speedup vs baseline: 7.9757x; 7.9757x over previous
import functools

import jax
import jax.numpy as jnp
from jax import lax
from jax.experimental import pallas as pl
from jax.experimental.pallas import tpu as pltpu
from jax.experimental.pallas import tpu_sc as plsc

B = 16384
F = 26
V = 100000
D = 16
OUT = 128

NC = 2
NS = 16
NW = NC * NS
ROWS_W = B // NW
CH = 128
IPC = CH * F
NVEC = IPC // 128
NCHUNK = ROWS_W // CH
VECS_W = ROWS_W * F // 128


def _sc_gather_body(idx_hbm, table_hbm, out_hbm, idx_v, rows_v, gsem):
    wid = lax.axis_index("s") * NC + lax.axis_index("c")
    vbase = wid * VECS_W

    def chunk(ci, carry):
        blk = wid * NCHUNK + ci
        r0 = vbase + ci * NVEC
        pltpu.sync_copy(idx_hbm.at[blk], idx_v)

        def fire(j, c):
            pltpu.async_copy(
                table_hbm.at[idx_v.at[j]],
                rows_v.at[pl.ds(j * 128, 128)],
                gsem,
            )
            return c

        lax.fori_loop(0, NVEC, fire, 0, unroll=False)
        pltpu.make_async_copy(
            out_hbm.at[pl.ds(r0 * 128, IPC)], rows_v, gsem
        ).wait()

        pltpu.sync_copy(rows_v, out_hbm.at[pl.ds(r0 * 128, IPC)])
        return carry

    lax.fori_loop(0, NCHUNK, chunk, 0, unroll=False)


_sc_gather = pl.kernel(
    _sc_gather_body,
    out_type=jax.ShapeDtypeStruct((B * F, D), jnp.float32),
    mesh=plsc.VectorSubcoreMesh(
        core_axis_name="c", subcore_axis_name="s", num_cores=NC, num_subcores=NS
    ),
    scratch_types=[
        pltpu.VMEM((NVEC, 128), jnp.int32),
        pltpu.VMEM((IPC, D), jnp.float32),
        pltpu.SemaphoreType.DMA,
    ],
    compiler_params=pltpu.CompilerParams(use_tc_tiling_on_sc=False),
)


BM = 2048


def _mm_body(a_ref, w_ref, b_ref, o_ref):
    acc = jnp.dot(a_ref[...], w_ref[...], preferred_element_type=jnp.float32)
    z = acc + b_ref[...]
    o_ref[...] = 1.0 / (1.0 + jnp.exp(-z))


_mm = pl.pallas_call(
    _mm_body,
    grid=(B // BM,),
    in_specs=[
        pl.BlockSpec((BM, F * D), lambda i: (i, 0)),
        pl.BlockSpec((F * D, OUT), lambda i: (0, 0)),
        pl.BlockSpec((1, OUT), lambda i: (0, 0)),
    ],
    out_specs=pl.BlockSpec((BM, OUT), lambda i: (i, 0)),
    out_shape=jax.ShapeDtypeStruct((B, OUT), jnp.float32),
)


def kernel(x, table, W, b):
    flat_idx = (x + (jnp.arange(F, dtype=jnp.int32) * V)[None, :]).reshape(
        NW * NCHUNK, NVEC, 128
    )
    flat_table = table.reshape(F * V, D)
    gathered = _sc_gather(flat_idx, flat_table)
    return _mm(gathered.reshape(B, F * D), W, b.reshape(1, OUT))


# speedup vs baseline: 39.0247x; 4.8930x over previous
import jax
import jax.numpy as jnp
from jax import lax
from jax.experimental import pallas as pl
from jax.experimental.pallas import tpu as pltpu
from jax.experimental.pallas import tpu_sc as plsc

B = 16384
F = 26
V = 100000
D = 16
OUT = 128


TR_VB = 100
VPAD = TR_VB * 1024
TR_NB = 1
TR_K = VPAD // TR_NB
NFG = (8, 8, 8, 2)


def _tr_body(t_ref, o_ref):
    x = t_ref[0]
    for m in range(TR_K // 1024):
        xs = [x[:, 1024 * m + 128 * k:1024 * m + 128 * (k + 1)]
              for k in range(8)]
        xh = jnp.concatenate(xs, axis=0)
        o_ref[128 * m:128 * (m + 1), :] = xh.T


def _make_tr(f0, nf):
    return pl.pallas_call(
        _tr_body,
        grid=(nf, TR_NB),
        in_specs=[pl.BlockSpec((1, 16, TR_K), lambda f, v: (f0 + f, 0, v))],
        out_specs=pl.BlockSpec(
            (TR_K // 8, 128), lambda f, v: (f * TR_NB + v, 0)
        ),
        out_shape=jax.ShapeDtypeStruct((nf * TR_VB * 128, 128), jnp.float32),
    )


_TRS = [_make_tr(8 * k, nf) for k, nf in enumerate(NFG)]


NC = 2
NS = 16
NW = NC * NS
ROWS_W = B // NW
CH = 128
NCHUNK = ROWS_W // CH
GV = 8
IPC = CH * GV
KBLK = B * 8


def _sc_g_body(idx_hbm, table_hbm, out_hbm, idx_v, rows_v, gsem):
    wid = lax.axis_index("s") * NC + lax.axis_index("c")

    def chunk(ci, carry):
        blk = wid * NCHUNK + ci
        pltpu.sync_copy(idx_hbm.at[blk], idx_v)

        def fire(j, c):
            pltpu.async_copy(
                table_hbm.at[idx_v.at[j]],
                rows_v.at[pl.ds(j * 128, 128)],
                gsem,
            )
            return c

        lax.fori_loop(0, GV, fire, 0, unroll=False)
        pltpu.make_async_copy(
            out_hbm.at[pl.ds(0, IPC)], rows_v, gsem
        ).wait()
        pltpu.sync_copy(rows_v, out_hbm.at[pl.ds(blk * IPC, IPC)])
        return carry

    lax.fori_loop(0, NCHUNK, chunk, 0, unroll=False)


_sc_g = pl.kernel(
    _sc_g_body,
    out_type=jax.ShapeDtypeStruct((KBLK, D), jnp.float32),
    mesh=plsc.VectorSubcoreMesh(
        core_axis_name="c", subcore_axis_name="s", num_cores=NC, num_subcores=NS
    ),
    scratch_types=[
        pltpu.VMEM((GV, 128), jnp.int32),
        pltpu.VMEM((IPC, D), jnp.float32),
        pltpu.SemaphoreType.DMA,
    ],
    compiler_params=pltpu.CompilerParams(use_tc_tiling_on_sc=False),
)


BM = 2048


def _mm_body(a0, a1, a2, a3, w_ref, b_ref, o_ref):
    acc = b_ref[...].astype(jnp.float32)
    for k, a in enumerate((a0, a1, a2, a3)):
        acc = acc + jnp.dot(
            a[...], w_ref[k], preferred_element_type=jnp.float32
        )
    o_ref[...] = 1.0 / (1.0 + jnp.exp(-acc))


_mm = pl.pallas_call(
    _mm_body,
    grid=(B // BM,),
    in_specs=[
        pl.BlockSpec((BM, 128), lambda i: (i, 0)),
        pl.BlockSpec((BM, 128), lambda i: (i, 0)),
        pl.BlockSpec((BM, 128), lambda i: (i, 0)),
        pl.BlockSpec((BM, 128), lambda i: (i, 0)),
        pl.BlockSpec((4, 128, OUT), lambda i: (0, 0, 0)),
        pl.BlockSpec((1, OUT), lambda i: (0, 0)),
    ],
    out_specs=pl.BlockSpec((BM, OUT), lambda i: (i, 0)),
    out_shape=jax.ShapeDtypeStruct((B, OUT), jnp.float32),
)


def _rowid(floc, xv):
    return (floc * TR_VB + xv // 1024) * 1024 + (xv % 128) * 8 + (
        xv // 128
    ) % 8


def kernel(x, table, W, b):
    tt = jnp.transpose(table, (0, 2, 1))
    floc8 = jnp.arange(8, dtype=jnp.int32)[None, :]

    outs = []
    for k in range(4):
        tab_k = _TRS[k](tt).reshape(NFG[k] * VPAD, D)
        if k < 3:
            rows = _rowid(floc8, x[:, 8 * k:8 * (k + 1)])
        else:
            xv = jnp.concatenate([x[:, 24:26], x[:, 0:6]], axis=1)
            floc = jnp.array([0, 1, 0, 0, 0, 0, 0, 0], jnp.int32)[None, :]
            rows = _rowid(floc, xv)
        idx_k = rows.reshape(NW * NCHUNK, GV * 128).reshape(
            NW * NCHUNK, GV, 128
        )
        outs.append(_sc_g(idx_k, tab_k).reshape(B, 128))

    wp = jnp.concatenate(
        [W, jnp.zeros((512 - F * D, OUT), jnp.float32)], axis=0
    ).reshape(4, 128, OUT)
    return _mm(*outs, wp, b.reshape(1, OUT))


# speedup vs baseline: 47.9780x; 1.2294x over previous
import jax
import jax.numpy as jnp
from jax import lax
from jax.experimental import pallas as pl
from jax.experimental.pallas import tpu as pltpu
from jax.experimental.pallas import tpu_sc as plsc

B = 16384
F = 26
V = 100000
D = 16
OUT = 128


TR_VB = 100
VPAD = TR_VB * 1024
TR_NB = 1
TR_K = VPAD // TR_NB
NFG = (16, 10)


def _tr_body(t_ref, o_ref):
    x = t_ref[0]
    for m in range(TR_K // 1024):
        xs = [x[:, 1024 * m + 128 * k:1024 * m + 128 * (k + 1)]
              for k in range(8)]
        xh = jnp.concatenate(xs, axis=0)
        o_ref[128 * m:128 * (m + 1), :] = xh.T


def _make_tr(f0, nf):
    return pl.pallas_call(
        _tr_body,
        grid=(nf, TR_NB),
        in_specs=[pl.BlockSpec((1, 16, TR_K), lambda f, v: (f0 + f, 0, v))],
        out_specs=pl.BlockSpec(
            (TR_K // 8, 128), lambda f, v: (f * TR_NB + v, 0)
        ),
        out_shape=jax.ShapeDtypeStruct((nf * TR_VB * 128, 128), jnp.float32),
    )


_TRS = [_make_tr(0, NFG[0]), _make_tr(NFG[0], NFG[1])]


NC = 2
NS = 16
NW = NC * NS
ROWS_W = B // NW
CH = 256
NCHUNK = ROWS_W // CH
GV = CH * 16 // 128
IPC = GV * 128
HALF = IPC // 2
KBLK = B * 8


def _sc_g_body(idx_hbm, table_hbm, out0_hbm, out1_hbm, idx_v, rows_v, gsem):
    wid = lax.axis_index("s") * NC + lax.axis_index("c")

    def chunk(ci, carry):
        blk = wid * NCHUNK + ci
        pltpu.sync_copy(idx_hbm.at[blk], idx_v)

        def fire(j, c):
            pltpu.async_copy(
                table_hbm.at[idx_v.at[j]],
                rows_v.at[pl.ds(j * 128, 128)],
                gsem,
            )
            return c

        lax.fori_loop(0, GV, fire, 0, unroll=False)
        pltpu.make_async_copy(
            out0_hbm.at[pl.ds(0, IPC)], rows_v, gsem
        ).wait()
        pltpu.sync_copy(
            rows_v.at[pl.ds(0, HALF)],
            out0_hbm.at[pl.ds(blk * HALF, HALF)],
        )
        pltpu.sync_copy(
            rows_v.at[pl.ds(HALF, HALF)],
            out1_hbm.at[pl.ds(blk * HALF, HALF)],
        )
        return carry

    lax.fori_loop(0, NCHUNK, chunk, 0, unroll=False)


_sc_g = pl.kernel(
    _sc_g_body,
    out_type=(
        jax.ShapeDtypeStruct((KBLK, D), jnp.float32),
        jax.ShapeDtypeStruct((KBLK, D), jnp.float32),
    ),
    mesh=plsc.VectorSubcoreMesh(
        core_axis_name="c", subcore_axis_name="s", num_cores=NC, num_subcores=NS
    ),
    scratch_types=[
        pltpu.VMEM((GV, 128), jnp.int32),
        pltpu.VMEM((IPC, D), jnp.float32),
        pltpu.SemaphoreType.DMA,
    ],
    compiler_params=pltpu.CompilerParams(use_tc_tiling_on_sc=False),
)


BM = 2048


def _mm_body(a0, a1, a2, a3, w_ref, b_ref, o_ref):
    acc = b_ref[...].astype(jnp.float32)
    for k, a in enumerate((a0, a1, a2, a3)):
        acc = acc + jnp.dot(
            a[...], w_ref[k], preferred_element_type=jnp.float32
        )
    o_ref[...] = 1.0 / (1.0 + jnp.exp(-acc))


_mm = pl.pallas_call(
    _mm_body,
    grid=(B // BM,),
    in_specs=[
        pl.BlockSpec((BM, 128), lambda i: (i, 0)),
        pl.BlockSpec((BM, 128), lambda i: (i, 0)),
        pl.BlockSpec((BM, 128), lambda i: (i, 0)),
        pl.BlockSpec((BM, 128), lambda i: (i, 0)),
        pl.BlockSpec((4, 128, OUT), lambda i: (0, 0, 0)),
        pl.BlockSpec((1, OUT), lambda i: (0, 0)),
    ],
    out_specs=pl.BlockSpec((BM, OUT), lambda i: (i, 0)),
    out_shape=jax.ShapeDtypeStruct((B, OUT), jnp.float32),
)


def _rowid(floc, xv):
    return (floc * TR_VB + xv // 1024) * 1024 + (xv % 128) * 8 + (
        xv // 128
    ) % 8


def _chunked(rows):
    return (
        rows.reshape(NW * NCHUNK, CH, 2, 8)
        .transpose(0, 2, 1, 3)
        .reshape(NW * NCHUNK, GV, 128)
    )


def kernel(x, table, W, b):
    tt = jnp.transpose(table, (0, 2, 1))
    floc8 = jnp.arange(8, dtype=jnp.int32)[None, :]

    tab_a = _TRS[0](tt).reshape(NFG[0] * VPAD, D)
    rows_a = jnp.concatenate(
        [_rowid(floc8, x[:, 0:8]), _rowid(floc8 + 8, x[:, 8:16])], axis=1
    )
    oa0, oa1 = _sc_g(_chunked(rows_a), tab_a)

    tab_b = _TRS[1](tt).reshape(NFG[1] * VPAD, D)
    flocb = jnp.array([8, 9, 0, 0, 0, 0, 0, 0], jnp.int32)[None, :]
    xvb = jnp.concatenate([x[:, 24:26], x[:, 0:6]], axis=1)
    rows_b = jnp.concatenate(
        [_rowid(floc8, x[:, 16:24]), _rowid(flocb, xvb)], axis=1
    )
    ob0, ob1 = _sc_g(_chunked(rows_b), tab_b)

    outs = [o.reshape(B, 128) for o in (oa0, oa1, ob0, ob1)]
    wp = jnp.concatenate(
        [W, jnp.zeros((512 - F * D, OUT), jnp.float32)], axis=0
    ).reshape(4, 128, OUT)
    return _mm(*outs, wp, b.reshape(1, OUT))
